# baseline (speedup 1.0000x reference)
import functools

import jax
import jax.numpy as jnp
from jax import lax
from jax.experimental import pallas as pl
from jax.experimental.pallas import tpu as pltpu
from jax.experimental.pallas import tpu_sc as plsc

R = 1024
V = 100000
B = 16384
L = 16
NC, NS = 2, 16
NW = NC * NS
ROWS_PER_W = R // NW
CH = 4096
NCH = B // CH


def _sc_body(t_hbm, idx_hbm, out_hbm, idx_v, row_v, obuf):
    cid = lax.axis_index("c")
    sid = lax.axis_index("s")
    wid = sid * NC + cid

    pltpu.sync_copy(idx_hbm, idx_v)

    def do_row(k, carry):
        r = wid * ROWS_PER_W + k
        pltpu.sync_copy(t_hbm.at[r], row_v)

        def do_chunk(ci, carry2):
            def do_vec(j, carry3):
                ids = idx_v[pl.ds(ci * CH + j * L, L)]
                obuf[pl.ds(j * L, L)] = plsc.load_gather(row_v, [ids])
                return carry3

            lax.fori_loop(0, CH // L, do_vec, 0, unroll=8)
            pltpu.sync_copy(obuf, out_hbm.at[r, pl.ds(ci * CH, CH)])
            return carry2

        lax.fori_loop(0, NCH, do_chunk, 0)
        return carry

    lax.fori_loop(0, ROWS_PER_W, do_row, 0)


@functools.partial(
    pl.kernel,
    mesh=plsc.VectorSubcoreMesh(core_axis_name="c", subcore_axis_name="s"),
    out_type=jax.ShapeDtypeStruct((R, B), jnp.float32),
    scratch_types=[
        pltpu.VMEM((B,), jnp.int32),
        pltpu.VMEM((V,), jnp.float32),
        pltpu.VMEM((CH,), jnp.float32),
    ],
    compiler_params=pltpu.CompilerParams(needs_layout_passes=False),
)
def _gather_cols(t_hbm, idx_hbm, out_hbm, idx_v, row_v, obuf):
    _sc_body(t_hbm, idx_hbm, out_hbm, idx_v, row_v, obuf)


def kernel(t, index):
    return _gather_cols(t, index.astype(jnp.int32))


# speedup vs baseline: 1.4513x; 1.4513x over previous
import functools

import jax
import jax.numpy as jnp
from jax import lax
from jax.experimental import pallas as pl
from jax.experimental.pallas import tpu as pltpu
from jax.experimental.pallas import tpu_sc as plsc

R = 1024
V = 100000
B = 16384
L = 16
NC, NS = 2, 16
NW = NC * NS
ROWS_PER_W = R // NW
CH = 4096
NCH = B // CH


def _sc_body(t_hbm, idx_hbm, out_hbm, idx_v, row_v, ob0, ob1, sem0, sem1):
    cid = lax.axis_index("c")
    sid = lax.axis_index("s")
    wid = sid * NC + cid
    row0 = wid * ROWS_PER_W
    ob = (ob0, ob1)
    sem = (sem0, sem1)

    pltpu.sync_copy(idx_hbm, idx_v)

    def do_row(k, carry):
        r = row0 + k
        pltpu.sync_copy(t_hbm.at[r], row_v)

        for ci in range(NCH):
            par = ci % 2
            if ci >= 2:
                pltpu.make_async_copy(
                    out_hbm.at[0, pl.ds(0, CH)], ob[par], sem[par]).wait()

            @plsc.parallel_loop(0, CH // L, unroll=8)
            def gather_chunk(j, par=par, ci=ci):
                ids = idx_v[pl.ds(ci * CH + j * L, L)]
                ob[par][pl.ds(j * L, L)] = plsc.load_gather(row_v, [ids])

            pltpu.make_async_copy(
                ob[par], out_hbm.at[r, pl.ds(ci * CH, CH)], sem[par]).start()
        pltpu.make_async_copy(out_hbm.at[0, pl.ds(0, CH)], ob0, sem0).wait()
        pltpu.make_async_copy(out_hbm.at[0, pl.ds(0, CH)], ob1, sem1).wait()
        return carry

    lax.fori_loop(0, ROWS_PER_W, do_row, 0)


@functools.partial(
    pl.kernel,
    mesh=plsc.VectorSubcoreMesh(core_axis_name="c", subcore_axis_name="s"),
    out_type=jax.ShapeDtypeStruct((R, B), jnp.float32),
    scratch_types=[
        pltpu.VMEM((B,), jnp.int32),
        pltpu.VMEM((V,), jnp.float32),
        pltpu.VMEM((CH,), jnp.float32),
        pltpu.VMEM((CH,), jnp.float32),
        pltpu.SemaphoreType.DMA,
        pltpu.SemaphoreType.DMA,
    ],
    compiler_params=pltpu.CompilerParams(needs_layout_passes=False),
)
def _gather_cols(*refs):
    _sc_body(*refs)


def kernel(t, index):
    return _gather_cols(t, index.astype(jnp.int32))


# speedup vs baseline: 6.9661x; 4.7999x over previous
import functools

import jax
import jax.numpy as jnp
from jax import lax
from jax.experimental import pallas as pl
from jax.experimental.pallas import tpu as pltpu
from jax.experimental.pallas import tpu_sc as plsc

R = 1024
V = 100000
B = 16384
NC, NS = 2, 16
NW = NC * NS
BW = B // NW
CK = 32
NCK = BW // CK


def _sc_body(tt_hbm, idx_hbm, out_hbm, idx_v, rb0, rb1, gs0, gs1, ws0, ws1):
    cid = lax.axis_index("c")
    sid = lax.axis_index("s")
    wid = sid * NC + cid
    base = wid * BW
    rb = (rb0, rb1)
    gsem = (gs0, gs1)
    wsem = (ws0, ws1)

    pltpu.sync_copy(idx_hbm.at[pl.ds(base, BW)], idx_v)

    def gather_start(c):
        pltpu.make_async_copy(
            tt_hbm.at[idx_v.at[pl.ds(c * CK, CK)]], rb[c % 2],
            gsem[c % 2]).start()

    def write_start(c):
        pltpu.make_async_copy(
            rb[c % 2], out_hbm.at[pl.ds(base + c * CK, CK)],
            wsem[c % 2]).start()

    gather_start(0)
    for c in range(NCK):
        par = c % 2
        pltpu.make_async_copy(
            tt_hbm.at[pl.ds(0, CK)], rb[par], gsem[par]).wait()
        if c + 1 < NCK:
            if c >= 1:
                pltpu.make_async_copy(
                    tt_hbm.at[pl.ds(0, CK)], rb[1 - par], wsem[1 - par]).wait()
            gather_start(c + 1)
        write_start(c)
    pltpu.make_async_copy(tt_hbm.at[pl.ds(0, CK)], rb0, wsem[0]).wait()
    pltpu.make_async_copy(tt_hbm.at[pl.ds(0, CK)], rb1, wsem[1]).wait()


@functools.partial(
    pl.kernel,
    mesh=plsc.VectorSubcoreMesh(core_axis_name="c", subcore_axis_name="s"),
    out_type=jax.ShapeDtypeStruct((B, R), jnp.float32),
    scratch_types=[
        pltpu.VMEM((BW,), jnp.int32),
        pltpu.VMEM((CK, R), jnp.float32),
        pltpu.VMEM((CK, R), jnp.float32),
        pltpu.SemaphoreType.DMA,
        pltpu.SemaphoreType.DMA,
        pltpu.SemaphoreType.DMA,
        pltpu.SemaphoreType.DMA,
    ],
    compiler_params=pltpu.CompilerParams(needs_layout_passes=False),
)
def _gather_rows(*refs):
    _sc_body(*refs)


def kernel(t, index):
    tt = t.T
    out_t = _gather_rows(tt, index.astype(jnp.int32))
    return out_t.T
